# baseline (speedup 1.0000x reference)
import jax
import jax.numpy as jnp
from jax.experimental import pallas as pl
from jax.experimental.pallas import tpu as pltpu

_BUFFER_SIZE = 4194304
_BUF_LEN = 4195840
_TAIL = _BUF_LEN - _BUFFER_SIZE

_NCHUNK = 16
_C = _BUFFER_SIZE // _NCHUNK


def _dma_body(x_hbm, buf_hbm, out_hbm, vbuf, vtail, sin, sout, stin, stout):
    def in_copy(i):
        return pltpu.make_async_copy(
            x_hbm.at[:, pl.ds(i * _C, _C)], vbuf.at[i], sin.at[i]
        )

    def out_copy(i):
        return pltpu.make_async_copy(
            vbuf.at[i], out_hbm.at[:, pl.ds(_TAIL + i * _C, _C)], sout.at[i]
        )

    tail_in = pltpu.make_async_copy(
        buf_hbm.at[:, pl.ds(_BUFFER_SIZE, _TAIL)], vtail, stin
    )
    tail_in.start()
    for i in range(_NCHUNK):
        in_copy(i).start()
    tail_in.wait()
    tail_out = pltpu.make_async_copy(vtail, out_hbm.at[:, pl.ds(0, _TAIL)], stout)
    tail_out.start()
    for i in range(_NCHUNK):
        in_copy(i).wait()
        out_copy(i).start()
    for i in range(_NCHUNK):
        out_copy(i).wait()
    tail_out.wait()


def kernel(x, buffer):
    return pl.pallas_call(
        _dma_body,
        out_shape=jax.ShapeDtypeStruct((1, _BUF_LEN), jnp.float32),
        in_specs=[
            pl.BlockSpec(memory_space=pltpu.MemorySpace.HBM),
            pl.BlockSpec(memory_space=pltpu.MemorySpace.HBM),
        ],
        out_specs=pl.BlockSpec(memory_space=pltpu.MemorySpace.HBM),
        scratch_shapes=[
            pltpu.VMEM((_NCHUNK, 1, _C), jnp.float32),
            pltpu.VMEM((1, _TAIL), jnp.float32),
            pltpu.SemaphoreType.DMA((_NCHUNK,)),
            pltpu.SemaphoreType.DMA((_NCHUNK,)),
            pltpu.SemaphoreType.DMA,
            pltpu.SemaphoreType.DMA,
        ],
    )(x, buffer)


# speedup vs baseline: 1.0249x; 1.0249x over previous
import jax
import jax.numpy as jnp
from jax.experimental import pallas as pl
from jax.experimental.pallas import tpu as pltpu

_BUFFER_SIZE = 4194304
_BUF_LEN = 4195840
_TAIL = _BUF_LEN - _BUFFER_SIZE

_NCHUNK = 8
_C = _BUFFER_SIZE // _NCHUNK


def _dma_body(x_hbm, buf_hbm, out_hbm, vbuf, vtail, sin, sout, stin, stout):
    def in_copy(i):
        return pltpu.make_async_copy(
            x_hbm.at[:, pl.ds(i * _C, _C)], vbuf.at[i], sin.at[i]
        )

    def out_copy(i):
        return pltpu.make_async_copy(
            vbuf.at[i], out_hbm.at[:, pl.ds(_TAIL + i * _C, _C)], sout.at[i]
        )

    tail_in = pltpu.make_async_copy(
        buf_hbm.at[:, pl.ds(_BUFFER_SIZE, _TAIL)], vtail, stin
    )
    tail_in.start()
    for i in range(_NCHUNK):
        in_copy(i).start()
    tail_in.wait()
    tail_out = pltpu.make_async_copy(vtail, out_hbm.at[:, pl.ds(0, _TAIL)], stout)
    tail_out.start()
    for i in range(_NCHUNK):
        in_copy(i).wait()
        out_copy(i).start()
    for i in range(_NCHUNK):
        out_copy(i).wait()
    tail_out.wait()


def kernel(x, buffer):
    return pl.pallas_call(
        _dma_body,
        out_shape=jax.ShapeDtypeStruct((1, _BUF_LEN), jnp.float32),
        in_specs=[
            pl.BlockSpec(memory_space=pltpu.MemorySpace.HBM),
            pl.BlockSpec(memory_space=pltpu.MemorySpace.HBM),
        ],
        out_specs=pl.BlockSpec(memory_space=pltpu.MemorySpace.HBM),
        scratch_shapes=[
            pltpu.VMEM((_NCHUNK, 1, _C), jnp.float32),
            pltpu.VMEM((1, _TAIL), jnp.float32),
            pltpu.SemaphoreType.DMA((_NCHUNK,)),
            pltpu.SemaphoreType.DMA((_NCHUNK,)),
            pltpu.SemaphoreType.DMA,
            pltpu.SemaphoreType.DMA,
        ],
    )(x, buffer)
